# Initial kernel scaffold; baseline (speedup 1.0000x reference)
import jax
import jax.numpy as jnp
from jax.experimental import pallas as pl


def kernel(embeddings, pos_table, mod_table, ln_gamma, ln_beta):
    raise NotImplementedError("write your pallas kernel here")


# speedup vs baseline: 2.0114x; 2.0114x over previous
import jax
import jax.numpy as jnp
from jax.experimental import pallas as pl
from jax.experimental.pallas import tpu as pltpu

_EPS = 1e-12


def _body(emb_ref, pos_ref, mod_ref, gamma_ref, beta_ref, out_ref):
    x = emb_ref[0] + pos_ref[...] + mod_ref[0]
    mu = jnp.mean(x, axis=-1, keepdims=True)
    xc = x - mu
    var = jnp.mean(xc * xc, axis=-1, keepdims=True)
    inv = jax.lax.rsqrt(var + _EPS)
    out_ref[0] = xc * inv * gamma_ref[...] + beta_ref[...]


def kernel(embeddings, pos_table, mod_table, ln_gamma, ln_beta):
    B, S, D = embeddings.shape
    BR = 256
    nj = S // BR
    j0 = 1024 // BR
    j1 = 2048 // BR

    mod3 = mod_table.reshape(3, 1, D)
    gamma2 = ln_gamma.reshape(1, D)
    beta2 = ln_beta.reshape(1, D)

    grid = (nj, B)

    return pl.pallas_call(
        _body,
        grid=grid,
        in_specs=[
            pl.BlockSpec((1, BR, D), lambda j, b: (b, j, 0)),
            pl.BlockSpec((BR, D), lambda j, b: (j, 0)),
            pl.BlockSpec(
                (1, 1, D),
                lambda j, b: ((j >= j0).astype(jnp.int32) + (j >= j1).astype(jnp.int32), 0, 0),
            ),
            pl.BlockSpec((1, D), lambda j, b: (0, 0)),
            pl.BlockSpec((1, D), lambda j, b: (0, 0)),
        ],
        out_specs=pl.BlockSpec((1, BR, D), lambda j, b: (b, j, 0)),
        out_shape=jax.ShapeDtypeStruct((B, S, D), embeddings.dtype),
        compiler_params=pltpu.CompilerParams(
            dimension_semantics=("arbitrary", "arbitrary"),
        ),
    )(embeddings, pos_table, mod3, gamma2, beta2)


# speedup vs baseline: 2.6238x; 1.3045x over previous
import jax
import jax.numpy as jnp
from jax.experimental import pallas as pl
from jax.experimental.pallas import tpu as pltpu

_EPS = 1e-12


def _body(emb_ref, pos_ref, mod_ref, gamma_ref, beta_ref, out_ref):
    x = emb_ref[0] + pos_ref[...] + mod_ref[0]
    mu = jnp.mean(x, axis=-1, keepdims=True)
    xc = x - mu
    var = jnp.mean(xc * xc, axis=-1, keepdims=True)
    inv = jax.lax.rsqrt(var + _EPS)
    out_ref[0] = xc * inv * gamma_ref[...] + beta_ref[...]


def kernel(embeddings, pos_table, mod_table, ln_gamma, ln_beta):
    B, S, D = embeddings.shape
    BR = 512
    nj = S // BR
    j0 = 1024 // BR
    j1 = 2048 // BR

    mod3 = mod_table.reshape(3, 1, D)
    gamma2 = ln_gamma.reshape(1, D)
    beta2 = ln_beta.reshape(1, D)

    grid = (nj, B)

    return pl.pallas_call(
        _body,
        grid=grid,
        in_specs=[
            pl.BlockSpec((1, BR, D), lambda j, b: (b, j, 0)),
            pl.BlockSpec((BR, D), lambda j, b: (j, 0)),
            pl.BlockSpec(
                (1, 1, D),
                lambda j, b: ((j >= j0).astype(jnp.int32) + (j >= j1).astype(jnp.int32), 0, 0),
            ),
            pl.BlockSpec((1, D), lambda j, b: (0, 0)),
            pl.BlockSpec((1, D), lambda j, b: (0, 0)),
        ],
        out_specs=pl.BlockSpec((1, BR, D), lambda j, b: (b, j, 0)),
        out_shape=jax.ShapeDtypeStruct((B, S, D), embeddings.dtype),
        compiler_params=pltpu.CompilerParams(
            dimension_semantics=("arbitrary", "arbitrary"),
        ),
    )(embeddings, pos_table, mod3, gamma2, beta2)


# speedup vs baseline: 2.9684x; 1.1313x over previous
import jax
import jax.numpy as jnp
from jax.experimental import pallas as pl
from jax.experimental.pallas import tpu as pltpu

_EPS = 1e-12


def _body(emb_ref, pos_ref, mod_ref, gamma_ref, beta_ref, out_ref):
    x = emb_ref[0] + pos_ref[...] + mod_ref[0]
    mu = jnp.mean(x, axis=-1, keepdims=True)
    xc = x - mu
    var = jnp.mean(xc * xc, axis=-1, keepdims=True)
    inv = jax.lax.rsqrt(var + _EPS)
    out_ref[0] = xc * inv * gamma_ref[...] + beta_ref[...]


def kernel(embeddings, pos_table, mod_table, ln_gamma, ln_beta):
    B, S, D = embeddings.shape
    BR = 1024
    nj = S // BR
    j0 = 1024 // BR
    j1 = 2048 // BR

    mod3 = mod_table.reshape(3, 1, D)
    gamma2 = ln_gamma.reshape(1, D)
    beta2 = ln_beta.reshape(1, D)

    grid = (nj, B)

    return pl.pallas_call(
        _body,
        grid=grid,
        in_specs=[
            pl.BlockSpec((1, BR, D), lambda j, b: (b, j, 0)),
            pl.BlockSpec((BR, D), lambda j, b: (j, 0)),
            pl.BlockSpec(
                (1, 1, D),
                lambda j, b: ((j >= j0).astype(jnp.int32) + (j >= j1).astype(jnp.int32), 0, 0),
            ),
            pl.BlockSpec((1, D), lambda j, b: (0, 0)),
            pl.BlockSpec((1, D), lambda j, b: (0, 0)),
        ],
        out_specs=pl.BlockSpec((1, BR, D), lambda j, b: (b, j, 0)),
        out_shape=jax.ShapeDtypeStruct((B, S, D), embeddings.dtype),
        compiler_params=pltpu.CompilerParams(
            dimension_semantics=("arbitrary", "arbitrary"),
        ),
    )(embeddings, pos_table, mod3, gamma2, beta2)
